# baseline (speedup 1.0000x reference)
import jax
import jax.numpy as jnp
from jax import lax
from jax.experimental import pallas as pl
from jax.experimental.pallas import tpu as pltpu
from jax.experimental.pallas import tpu_sc as plsc

CLASSES = 1000
OUT_W = CLASSES + 1
B, L = 4096, 20
N = B * L

NUM_CORES = 2
NUM_SUBCORES = 16
NW = NUM_CORES * NUM_SUBCORES
ROWS_PER_W = N // NW
R = 32
NCH = ROWS_PER_W // R
G = R // 16
BLK = R * OUT_W


def _sc_body(acts_hbm, durs_hbm, out_hbm, acts_v, durs_v, buf0, buf1, sem0, sem1):
    wid = lax.axis_index("s") * NUM_CORES + lax.axis_index("c")
    base0 = wid * ROWS_PER_W

    pltpu.sync_copy(acts_hbm.at[pl.ds(base0, ROWS_PER_W)], acts_v)
    pltpu.sync_copy(durs_hbm.at[pl.ds(base0, ROWS_PER_W)], durs_v)

    zeros16 = jnp.zeros((16,), jnp.float32)
    ones16 = jnp.ones((16,), jnp.float32)
    iota16 = lax.iota(jnp.int32, 16)

    def zero_blk(i, _):
        off = i * (7 * 16)
        for j in range(7):
            buf0[pl.ds(off + j * 16, 16)] = zeros16
            buf1[pl.ds(off + j * 16, 16)] = zeros16
        return 0

    lax.fori_loop(0, BLK // (7 * 16), zero_blk, 0)

    def fill(buf, sem, chunk):
        off = chunk * R
        for g in range(G):
            a16 = acts_v[pl.ds(off + g * 16, 16)]
            d16 = durs_v[pl.ds(off + g * 16, 16)]
            rowbase = (iota16 + g * 16) * OUT_W
            plsc.store_scatter(buf, [rowbase + a16], ones16)
            plsc.store_scatter(buf, [rowbase + CLASSES], d16)
        pltpu.make_async_copy(
            buf, out_hbm.at[pl.ds((base0 + off) * OUT_W, BLK)], sem
        ).start()

    def clean(buf, sem, chunk):
        off = chunk * R
        pltpu.make_async_copy(
            buf, out_hbm.at[pl.ds((base0 + off) * OUT_W, BLK)], sem
        ).wait()
        for g in range(G):
            a16 = acts_v[pl.ds(off + g * 16, 16)]
            rowbase = (iota16 + g * 16) * OUT_W
            plsc.store_scatter(buf, [rowbase + a16], zeros16)

    fill(buf0, sem0, 0)
    fill(buf1, sem1, 1)

    def body(p, _):
        c0 = 2 * p
        clean(buf0, sem0, c0 - 2)
        fill(buf0, sem0, c0)
        clean(buf1, sem1, c0 - 1)
        fill(buf1, sem1, c0 + 1)
        return 0

    lax.fori_loop(1, NCH // 2, body, 0)

    pltpu.make_async_copy(
        buf0, out_hbm.at[pl.ds((base0 + (NCH - 2) * R) * OUT_W, BLK)], sem0
    ).wait()
    pltpu.make_async_copy(
        buf1, out_hbm.at[pl.ds((base0 + (NCH - 1) * R) * OUT_W, BLK)], sem1
    ).wait()


_sc_call = pl.kernel(
    _sc_body,
    out_type=jax.ShapeDtypeStruct((N * OUT_W,), jnp.float32),
    mesh=plsc.VectorSubcoreMesh(core_axis_name="c", subcore_axis_name="s"),
    scratch_types=[
        pltpu.VMEM((ROWS_PER_W,), jnp.int32),
        pltpu.VMEM((ROWS_PER_W,), jnp.float32),
        pltpu.VMEM((BLK,), jnp.float32),
        pltpu.VMEM((BLK,), jnp.float32),
        pltpu.SemaphoreType.DMA,
        pltpu.SemaphoreType.DMA,
    ],
    compiler_params=pltpu.CompilerParams(
        use_tc_tiling_on_sc=False, needs_layout_passes=False
    ),
)


def kernel(x):
    acts = x[..., 0].astype(jnp.int32).reshape(N)
    durs = x[..., 1].reshape(N)
    out = _sc_call(acts, durs)
    return out.reshape(B, L, OUT_W)


# speedup vs baseline: 12.3528x; 12.3528x over previous
import jax
import jax.numpy as jnp
from jax import lax
from jax.experimental import pallas as pl
from jax.experimental.pallas import tpu as pltpu
from jax.experimental.pallas import tpu_sc as plsc

CLASSES = 1000
OUT_W = CLASSES + 1
B, L = 4096, 20
CT = 126
NBT = 32
H = 3
CPH = CT // H
S = L * CT


def _sc_body(xin_hbm, out_hbm, st0, st1, st2, b0, b1, b2, sa, s0, s1, s2):
    w = lax.axis_index("s") * 2 + lax.axis_index("c")
    zeros16 = jnp.zeros((16,), jnp.float32)
    ones16 = jnp.ones((16,), jnp.float32)
    iota16 = lax.iota(jnp.int32, 16)
    bufs = (b0, b1, b2)
    sems = (s0, s1, s2)
    stages = (st0, st1, st2)

    def zrow(i, _):
        ct = i >> 3
        cr = i & 7
        for buf in bufs:
            for j in range(8):
                buf[ct, cr, pl.ds(j * 16, 16)] = zeros16
        return 0

    lax.fori_loop(0, CPH * 8, zrow, 0)

    def stage_start(l, slot):
        pltpu.make_async_copy(
            xin_hbm.at[l, :, pl.ds(w * 128, 128)], stages[slot], sa
        ).start()

    def stage_wait(slot):
        pltpu.make_async_copy(
            xin_hbm.at[0, :, pl.ds(w * 128, 128)], stages[slot], sa
        ).wait()

    def scan(buf, st, h, vals):
        lo = h * CPH
        for j in range(8):
            a16 = st[0, pl.ds(j * 16, 16)]
            ctl = (a16 >> 3) - lo
            ok = (ctl >= 0) & (ctl < CPH)
            ctl = jnp.minimum(jnp.maximum(ctl, 0), CPH - 1)
            plsc.store_scatter(
                buf, [ctl, a16 & 7, iota16 + j * 16], vals, mask=ok
            )

    def fill(h, l, slot):
        buf = bufs[h]
        st = stages[slot]
        scan(buf, st, h, ones16)
        if h == H - 1:
            for j in range(8):
                d16 = plsc.bitcast(st[1, pl.ds(j * 16, 16)], jnp.float32)
                buf[CPH - 1, 0, pl.ds(j * 16, 16)] = d16
        pltpu.make_async_copy(
            buf, out_hbm.at[pl.ds(l * CT + h * CPH, CPH), w], sems[h]
        ).start()

    def clean(h, l_old, slot_old):
        buf = bufs[h]
        pltpu.make_async_copy(
            buf, out_hbm.at[pl.ds(l_old * CT + h * CPH, CPH), w], sems[h]
        ).wait()
        scan(buf, stages[slot_old], h, zeros16)

    stage_start(0, 0)
    stage_wait(0)
    stage_start(1, 1)
    for h in range(H):
        fill(h, 0, 0)

    def step(l, slot, prefetch):
        stage_wait(slot)
        if prefetch:
            stage_start(l + 1, (slot + 1) % 3)
        for h in range(H):
            clean(h, l - 1, (slot + 2) % 3)
            fill(h, l, slot)

    def body(p, _):
        l = 3 * p + 1
        step(l, 1, True)
        step(l + 1, 2, True)
        step(l + 2, 0, True)
        return 0

    lax.fori_loop(0, (L - 2) // 3, body, 0)
    step(L - 1, (L - 1) % 3, False)

    for h in range(H):
        pltpu.make_async_copy(
            bufs[h], out_hbm.at[pl.ds((L - 1) * CT + h * CPH, CPH), w], sems[h]
        ).wait()


_sc_call = pl.kernel(
    _sc_body,
    out_type=jax.ShapeDtypeStruct((S, NBT, 8, 128), jnp.float32),
    mesh=plsc.VectorSubcoreMesh(core_axis_name="c", subcore_axis_name="s"),
    scratch_types=[
        pltpu.VMEM((2, 128), jnp.int32),
        pltpu.VMEM((2, 128), jnp.int32),
        pltpu.VMEM((2, 128), jnp.int32),
        pltpu.VMEM((CPH, 8, 128), jnp.float32),
        pltpu.VMEM((CPH, 8, 128), jnp.float32),
        pltpu.VMEM((CPH, 8, 128), jnp.float32),
        pltpu.SemaphoreType.DMA,
        pltpu.SemaphoreType.DMA,
        pltpu.SemaphoreType.DMA,
        pltpu.SemaphoreType.DMA,
    ],
    compiler_params=pltpu.CompilerParams(
        use_tc_tiling_on_sc=False, needs_layout_passes=False
    ),
)


def kernel(x):
    acts = x[..., 0].astype(jnp.int32).T
    durs = lax.bitcast_convert_type(x[..., 1], jnp.int32).T
    xin = jnp.stack([acts, durs], axis=1)
    y = _sc_call(xin)
    y5 = y.reshape(L, CT, NBT, 8, 128)
    out = y5.transpose(2, 4, 0, 1, 3).reshape(B, L, CT * 8)[:, :, :OUT_W]
    return out


# speedup vs baseline: 12.6763x; 1.0262x over previous
import jax
import jax.numpy as jnp
from jax import lax
from jax.experimental import pallas as pl
from jax.experimental.pallas import tpu as pltpu
from jax.experimental.pallas import tpu_sc as plsc

CLASSES = 1000
OUT_W = CLASSES + 1
B, L = 4096, 20
CT = 126
NBT = 32
H = 3
CPH = CT // H
S = L * CT


def _sc_body(xin_hbm, out_hbm, st0, st1, st2, b0, b1, b2, sa, s0, s1, s2):
    w = lax.axis_index("s") * 2 + lax.axis_index("c")
    zeros16 = jnp.zeros((16,), jnp.float32)
    ones16 = jnp.ones((16,), jnp.float32)
    iota16 = lax.iota(jnp.int32, 16)
    bufs = (b0, b1, b2)
    sems = (s0, s1, s2)
    stages = (st0, st1, st2)

    def stage_start(l, slot):
        pltpu.make_async_copy(xin_hbm.at[l, w], stages[slot], sa).start()

    def stage_wait(slot):
        pltpu.make_async_copy(xin_hbm.at[0, w], stages[slot], sa).wait()

    def scan(buf, st, h, vals):
        lo = h * CPH
        for j in range(8):
            a16 = plsc.bitcast(st[0, pl.ds(j * 16, 16)], jnp.float32).astype(
                jnp.int32
            )
            ctl = (a16 >> 3) - lo
            ok = (ctl >= 0) & (ctl < CPH)
            ctl = jnp.minimum(jnp.maximum(ctl, 0), CPH - 1)
            plsc.store_scatter(
                buf, [ctl, a16 & 7, iota16 + j * 16], vals, mask=ok
            )

    def fill(h, l, slot):
        buf = bufs[h]
        st = stages[slot]
        scan(buf, st, h, ones16)
        if h == H - 1:
            for j in range(8):
                d16 = plsc.bitcast(st[1, pl.ds(j * 16, 16)], jnp.float32)
                buf[CPH - 1, 0, pl.ds(j * 16, 16)] = d16
        pltpu.make_async_copy(
            buf, out_hbm.at[pl.ds(l * CT + h * CPH, CPH), w], sems[h]
        ).start()

    def clean(h, l_old, slot_old):
        buf = bufs[h]
        pltpu.make_async_copy(
            buf, out_hbm.at[pl.ds(l_old * CT + h * CPH, CPH), w], sems[h]
        ).wait()
        scan(buf, stages[slot_old], h, zeros16)

    stage_start(0, 0)

    def zero_blk(buf):
        def zrow(i, _):
            ct = i >> 3
            cr = i & 7
            for j in range(8):
                buf[ct, cr, pl.ds(j * 16, 16)] = zeros16
            return 0

        lax.fori_loop(0, CPH * 8, zrow, 0)

    zero_blk(b0)
    stage_wait(0)
    stage_start(1, 1)
    fill(0, 0, 0)
    zero_blk(b1)
    fill(1, 0, 0)
    zero_blk(b2)
    fill(2, 0, 0)

    def step(l, slot, prefetch):
        stage_wait(slot)
        if prefetch:
            stage_start(l + 1, (slot + 1) % 3)
        for h in range(H):
            clean(h, l - 1, (slot + 2) % 3)
            fill(h, l, slot)

    def body(p, _):
        l = 3 * p + 1
        step(l, 1, True)
        step(l + 1, 2, True)
        step(l + 2, 0, True)
        return 0

    lax.fori_loop(0, (L - 2) // 3, body, 0)
    step(L - 1, (L - 1) % 3, False)

    for h in range(H):
        pltpu.make_async_copy(
            bufs[h], out_hbm.at[pl.ds((L - 1) * CT + h * CPH, CPH), w], sems[h]
        ).wait()


_sc_call = pl.kernel(
    _sc_body,
    out_type=jax.ShapeDtypeStruct((S, NBT, 8, 128), jnp.float32),
    mesh=plsc.VectorSubcoreMesh(core_axis_name="c", subcore_axis_name="s"),
    scratch_types=[
        pltpu.VMEM((2, 128), jnp.int32),
        pltpu.VMEM((2, 128), jnp.int32),
        pltpu.VMEM((2, 128), jnp.int32),
        pltpu.VMEM((CPH, 8, 128), jnp.float32),
        pltpu.VMEM((CPH, 8, 128), jnp.float32),
        pltpu.VMEM((CPH, 8, 128), jnp.float32),
        pltpu.SemaphoreType.DMA,
        pltpu.SemaphoreType.DMA,
        pltpu.SemaphoreType.DMA,
        pltpu.SemaphoreType.DMA,
    ],
    compiler_params=pltpu.CompilerParams(
        use_tc_tiling_on_sc=False, needs_layout_passes=False
    ),
)


def kernel(x):
    xin = (
        lax.bitcast_convert_type(x, jnp.int32)
        .reshape(NBT, 128, L, 2)
        .transpose(2, 0, 3, 1)
    )
    y = _sc_call(xin)
    y5 = y.reshape(L, CT, NBT, 8, 128)
    out = y5.transpose(2, 4, 0, 1, 3).reshape(B, L, CT * 8)[:, :, :OUT_W]
    return out


# speedup vs baseline: 12.8741x; 1.0156x over previous
import jax
import jax.numpy as jnp
from jax import lax
from jax.experimental import pallas as pl
from jax.experimental.pallas import tpu as pltpu
from jax.experimental.pallas import tpu_sc as plsc

CLASSES = 1000
OUT_W = CLASSES + 1
B, L = 4096, 20
CT = 126
NBT = 32
H = 2
CPH = CT // H
S = L * CT


def _sc_body(xin_hbm, out_hbm, st0, st1, st2, b0, b1, sa, s0, s1):
    w = lax.axis_index("s") * 2 + lax.axis_index("c")
    zeros16 = jnp.zeros((16,), jnp.float32)
    ones16 = jnp.ones((16,), jnp.float32)
    iota16 = lax.iota(jnp.int32, 16)
    bufs = (b0, b1)
    sems = (s0, s1)
    stages = (st0, st1, st2)

    def stage_start(l, slot):
        pltpu.make_async_copy(xin_hbm.at[l, w], stages[slot], sa).start()

    def stage_wait(slot):
        pltpu.make_async_copy(xin_hbm.at[0, w], stages[slot], sa).wait()

    def scan(buf, st, h, vals):
        lo = h * CPH
        for j in range(8):
            a16 = plsc.bitcast(st[0, pl.ds(j * 16, 16)], jnp.float32).astype(
                jnp.int32
            )
            ctl = (a16 >> 3) - lo
            ok = (ctl >= 0) & (ctl < CPH)
            ctl = jnp.minimum(jnp.maximum(ctl, 0), CPH - 1)
            plsc.store_scatter(
                buf, [ctl, a16 & 7, iota16 + j * 16], vals, mask=ok
            )

    def fill(h, l, slot):
        buf = bufs[h]
        st = stages[slot]
        scan(buf, st, h, ones16)
        if h == H - 1:
            for j in range(8):
                d16 = plsc.bitcast(st[1, pl.ds(j * 16, 16)], jnp.float32)
                buf[CPH - 1, 0, pl.ds(j * 16, 16)] = d16
        pltpu.make_async_copy(
            buf, out_hbm.at[pl.ds(l * CT + h * CPH, CPH), w], sems[h]
        ).start()

    def clean(h, l_old, slot_old):
        buf = bufs[h]
        pltpu.make_async_copy(
            buf, out_hbm.at[pl.ds(l_old * CT + h * CPH, CPH), w], sems[h]
        ).wait()
        scan(buf, stages[slot_old], h, zeros16)

    stage_start(0, 0)

    def zero_blk(buf):
        def zrow(i, _):
            ct = i >> 3
            cr = i & 7
            for j in range(8):
                buf[ct, cr, pl.ds(j * 16, 16)] = zeros16
            return 0

        lax.fori_loop(0, CPH * 8, zrow, 0)

    zero_blk(b0)
    stage_wait(0)
    stage_start(1, 1)
    fill(0, 0, 0)
    zero_blk(b1)
    fill(1, 0, 0)

    def step(l, slot, prefetch):
        stage_wait(slot)
        if prefetch:
            stage_start(l + 1, (slot + 1) % 3)
        for h in range(H):
            clean(h, l - 1, (slot + 2) % 3)
            fill(h, l, slot)

    def body(p, _):
        l = 3 * p + 1
        step(l, 1, True)
        step(l + 1, 2, True)
        step(l + 2, 0, True)
        return 0

    lax.fori_loop(0, (L - 2) // 3, body, 0)
    step(L - 1, (L - 1) % 3, False)

    for h in range(H):
        pltpu.make_async_copy(
            bufs[h], out_hbm.at[pl.ds((L - 1) * CT + h * CPH, CPH), w], sems[h]
        ).wait()


_sc_call = pl.kernel(
    _sc_body,
    out_type=jax.ShapeDtypeStruct((S, NBT, 8, 128), jnp.float32),
    mesh=plsc.VectorSubcoreMesh(core_axis_name="c", subcore_axis_name="s"),
    scratch_types=[
        pltpu.VMEM((2, 128), jnp.int32),
        pltpu.VMEM((2, 128), jnp.int32),
        pltpu.VMEM((2, 128), jnp.int32),
        pltpu.VMEM((CPH, 8, 128), jnp.float32),
        pltpu.VMEM((CPH, 8, 128), jnp.float32),
        pltpu.SemaphoreType.DMA,
        pltpu.SemaphoreType.DMA,
        pltpu.SemaphoreType.DMA,
    ],
    compiler_params=pltpu.CompilerParams(
        use_tc_tiling_on_sc=False, needs_layout_passes=False
    ),
)


def kernel(x):
    xin = (
        lax.bitcast_convert_type(x, jnp.int32)
        .reshape(NBT, 128, L, 2)
        .transpose(2, 0, 3, 1)
    )
    y = _sc_call(xin)
    y5 = y.reshape(L, CT, NBT, 8, 128)
    out = y5.transpose(2, 4, 0, 1, 3).reshape(B, L, CT * 8)[:, :, :OUT_W]
    return out
